# baseline (speedup 1.0000x reference)
import functools

import jax
import jax.numpy as jnp
from jax.experimental import pallas as pl
from jax.experimental.pallas import tpu as pltpu

_N = 50000
_E = 1024
_D = 64
_NBLK = 2000


def _body(x_ref, h_ref, o_ref, acc_ref, *, nsteps):
    i = pl.program_id(0)

    @pl.when(i == 0)
    def _init():
        acc_ref[...] = jnp.zeros_like(acc_ref)

    mask = (h_ref[...] > 0).astype(jnp.float32)
    acc_ref[...] += jax.lax.dot_general(
        mask, x_ref[...],
        dimension_numbers=(((0,), (0,)), ((), ())),
        preferred_element_type=jnp.float32,
    )

    @pl.when(i == nsteps - 1)
    def _fin():
        acc = acc_ref[...]
        means = acc / acc[:, _D:_D + 1]
        o_ref[...] = jnp.max(means, axis=0, keepdims=True)


def kernel(node_embeddings, hypergraph_matrix):
    n, d = node_embeddings.shape
    e = hypergraph_matrix.shape[1]
    x_aug = jnp.zeros((n, 128), dtype=jnp.float32)
    x_aug = x_aug.at[:, :d].set(node_embeddings)
    x_aug = x_aug.at[:, d].set(1.0)

    nsteps = n // _NBLK
    out = pl.pallas_call(
        functools.partial(_body, nsteps=nsteps),
        grid=(nsteps,),
        in_specs=[
            pl.BlockSpec((_NBLK, 128), lambda i: (i, 0)),
            pl.BlockSpec((_NBLK, e), lambda i: (i, 0)),
        ],
        out_specs=pl.BlockSpec((1, 128), lambda i: (0, 0)),
        out_shape=jax.ShapeDtypeStruct((1, 128), jnp.float32),
        scratch_shapes=[pltpu.VMEM((e, 128), jnp.float32)],
        compiler_params=pltpu.CompilerParams(
            dimension_semantics=("arbitrary",),
        ),
    )(x_aug, hypergraph_matrix)
    return out[0, :d]


# speedup vs baseline: 1.9947x; 1.9947x over previous
import functools

import jax
import jax.numpy as jnp
from jax.experimental import pallas as pl
from jax.experimental.pallas import tpu as pltpu

_NBLK = 2000


def _body(x_ref, h_ref, o_ref, acc_ref, cnt_ref, *, nsteps):
    i = pl.program_id(0)

    @pl.when(i == 0)
    def _init():
        acc_ref[...] = jnp.zeros_like(acc_ref)
        cnt_ref[...] = jnp.zeros_like(cnt_ref)

    mask = (h_ref[...] > 0).astype(jnp.float32)
    dn = (((0,), (0,)), ((), ()))
    acc_ref[...] += jax.lax.dot_general(
        mask, x_ref[...], dimension_numbers=dn,
        preferred_element_type=jnp.float32)
    ones = jnp.ones((mask.shape[0], 8), jnp.float32)
    cnt_ref[...] += jax.lax.dot_general(
        mask, ones, dimension_numbers=dn,
        preferred_element_type=jnp.float32)

    @pl.when(i == nsteps - 1)
    def _fin():
        means = acc_ref[...] / cnt_ref[:, 0:1]
        o_ref[...] = jnp.max(means, axis=0, keepdims=True)


def kernel(node_embeddings, hypergraph_matrix):
    n, d = node_embeddings.shape
    e = hypergraph_matrix.shape[1]
    nsteps = n // _NBLK
    out = pl.pallas_call(
        functools.partial(_body, nsteps=nsteps),
        grid=(nsteps,),
        in_specs=[
            pl.BlockSpec((_NBLK, d), lambda i: (i, 0)),
            pl.BlockSpec((_NBLK, e), lambda i: (i, 0)),
        ],
        out_specs=pl.BlockSpec((1, d), lambda i: (0, 0)),
        out_shape=jax.ShapeDtypeStruct((1, d), jnp.float32),
        scratch_shapes=[
            pltpu.VMEM((e, d), jnp.float32),
            pltpu.VMEM((e, 8), jnp.float32),
        ],
        compiler_params=pltpu.CompilerParams(
            dimension_semantics=("arbitrary",),
        ),
    )(node_embeddings, hypergraph_matrix)
    return out[0]


# speedup vs baseline: 2.0295x; 1.0175x over previous
import functools

import jax
import jax.numpy as jnp
from jax.experimental import pallas as pl
from jax.experimental.pallas import tpu as pltpu

_NBLK = 2000


def _body(x_ref, h_ref, o_ref, acc_ref, *, nsteps, d):
    i = pl.program_id(0)

    @pl.when(i == 0)
    def _init():
        acc_ref[...] = jnp.zeros_like(acc_ref)

    nblk = x_ref.shape[0]
    mask = (h_ref[...] > 0).astype(jnp.float32).astype(jnp.bfloat16)

    x = x_ref[...]
    xhi = x.astype(jnp.bfloat16)
    xlo = (x - xhi.astype(jnp.float32)).astype(jnp.bfloat16)
    col0 = (jax.lax.broadcasted_iota(jnp.int32, (nblk, d), 1) == 0)
    ones_col = col0.astype(jnp.float32).astype(jnp.bfloat16)
    zero_pad = jnp.zeros((nblk, d), jnp.bfloat16)
    xa_hi = jnp.concatenate([xhi, zero_pad], axis=1)
    xa_lo = jnp.concatenate([xlo, ones_col], axis=1)

    dn = (((0,), (0,)), ((), ()))
    acc_ref[...] += (
        jax.lax.dot_general(mask, xa_hi, dimension_numbers=dn,
                            preferred_element_type=jnp.float32)
        + jax.lax.dot_general(mask, xa_lo, dimension_numbers=dn,
                              preferred_element_type=jnp.float32)
    )

    @pl.when(i == nsteps - 1)
    def _fin():
        acc = acc_ref[...]
        means = acc[:, :d] / acc[:, d:d + 1]
        o_ref[...] = jnp.max(means, axis=0, keepdims=True)


def kernel(node_embeddings, hypergraph_matrix):
    n, d = node_embeddings.shape
    e = hypergraph_matrix.shape[1]
    nsteps = n // _NBLK
    out = pl.pallas_call(
        functools.partial(_body, nsteps=nsteps, d=d),
        grid=(nsteps,),
        in_specs=[
            pl.BlockSpec((_NBLK, d), lambda i: (i, 0)),
            pl.BlockSpec((_NBLK, e), lambda i: (i, 0)),
        ],
        out_specs=pl.BlockSpec((1, d), lambda i: (0, 0)),
        out_shape=jax.ShapeDtypeStruct((1, d), jnp.float32),
        scratch_shapes=[
            pltpu.VMEM((e, 2 * d), jnp.float32),
        ],
        compiler_params=pltpu.CompilerParams(
            dimension_semantics=("arbitrary",),
        ),
    )(node_embeddings, hypergraph_matrix)
    return out[0]


# speedup vs baseline: 2.1253x; 1.0472x over previous
import functools

import jax
import jax.numpy as jnp
from jax.experimental import pallas as pl
from jax.experimental.pallas import tpu as pltpu

_NBLK = 5000


def _body(x_ref, h_ref, o_ref, acc_ref, *, nsteps, d):
    i = pl.program_id(0)

    @pl.when(i == 0)
    def _init():
        acc_ref[...] = jnp.zeros_like(acc_ref)

    nblk = x_ref.shape[0]
    mask = (h_ref[...] > 0).astype(jnp.float32).astype(jnp.bfloat16)

    x = x_ref[...]
    xhi = x.astype(jnp.bfloat16)
    xlo = (x - xhi.astype(jnp.float32)).astype(jnp.bfloat16)
    col0 = (jax.lax.broadcasted_iota(jnp.int32, (nblk, d), 1) == 0)
    ones_col = col0.astype(jnp.float32).astype(jnp.bfloat16)
    zero_pad = jnp.zeros((nblk, d), jnp.bfloat16)
    xa_hi = jnp.concatenate([xhi, zero_pad], axis=1)
    xa_lo = jnp.concatenate([xlo, ones_col], axis=1)

    dn = (((0,), (0,)), ((), ()))
    acc_ref[...] += (
        jax.lax.dot_general(mask, xa_hi, dimension_numbers=dn,
                            preferred_element_type=jnp.float32)
        + jax.lax.dot_general(mask, xa_lo, dimension_numbers=dn,
                              preferred_element_type=jnp.float32)
    )

    @pl.when(i == nsteps - 1)
    def _fin():
        acc = acc_ref[...]
        means = acc[:, :d] / acc[:, d:d + 1]
        o_ref[...] = jnp.max(means, axis=0, keepdims=True)


def kernel(node_embeddings, hypergraph_matrix):
    n, d = node_embeddings.shape
    e = hypergraph_matrix.shape[1]
    nsteps = n // _NBLK
    out = pl.pallas_call(
        functools.partial(_body, nsteps=nsteps, d=d),
        grid=(nsteps,),
        in_specs=[
            pl.BlockSpec((_NBLK, d), lambda i: (i, 0)),
            pl.BlockSpec((_NBLK, e), lambda i: (i, 0)),
        ],
        out_specs=pl.BlockSpec((1, d), lambda i: (0, 0)),
        out_shape=jax.ShapeDtypeStruct((1, d), jnp.float32),
        scratch_shapes=[
            pltpu.VMEM((e, 2 * d), jnp.float32),
        ],
        compiler_params=pltpu.CompilerParams(
            dimension_semantics=("arbitrary",),
            vmem_limit_bytes=100 * 1024 * 1024,
        ),
    )(node_embeddings, hypergraph_matrix)
    return out[0]
